# baseline (speedup 1.0000x reference)
import functools

import jax
import jax.numpy as jnp
from jax import lax
from jax.experimental import pallas as pl
from jax.experimental.pallas import tpu as pltpu
from jax.experimental.pallas import tpu_sc as plsc

_NC, _NS, _L = 2, 16, 16
_NW = _NC * _NS
_R = 8


def kernel(tokens, emb):
    B, N, C = tokens.shape
    rows_w = N // _NW
    n_chunks = rows_w // _R
    W = _R * C

    tok_flat = tokens.reshape(B, N * C)
    emb_flat = emb.reshape(-1)

    mesh = plsc.VectorSubcoreMesh(
        core_axis_name="c", subcore_axis_name="s",
        num_cores=_NC, num_subcores=_NS)

    @functools.partial(
        pl.kernel,
        out_type=jax.ShapeDtypeStruct((B, N * C), jnp.float32),
        mesh=mesh,
        scratch_types=[
            pltpu.VMEM((W,), jnp.float32),
            pltpu.VMEM((W,), jnp.float32),
        ],
    )
    def sc_add(tok_hbm, emb_hbm, out_hbm, emb_v, tok_v):
        wid = lax.axis_index("s") * _NC + lax.axis_index("c")
        base = wid * (rows_w * C)

        def chunk_body(ci, carry):
            off = base + ci * W
            pltpu.sync_copy(emb_hbm.at[pl.ds(off, W)], emb_v)
            for b in range(B):
                pltpu.sync_copy(tok_hbm.at[b, pl.ds(off, W)], tok_v)

                def add_body(i, c2):
                    s = pl.ds(i * _L, _L)
                    plsc.addupdate(tok_v.at[s], emb_v[s])
                    return c2

                lax.fori_loop(0, W // _L, add_body, 0)
                pltpu.sync_copy(tok_v, out_hbm.at[b, pl.ds(off, W)])
            return carry

        lax.fori_loop(0, n_chunks, chunk_body, 0)

    out = sc_add(tok_flat, emb_flat)
    return out.reshape(B, N, C)


# speedup vs baseline: 1.7215x; 1.7215x over previous
import functools

import jax
import jax.numpy as jnp
from jax import lax
from jax.experimental import pallas as pl
from jax.experimental.pallas import tpu as pltpu
from jax.experimental.pallas import tpu_sc as plsc

_NC, _NS, _L = 2, 16, 16
_NW = _NC * _NS
_R = 8


def kernel(tokens, emb):
    B, N, C = tokens.shape
    rows_w = N // _NW
    n_chunks = rows_w // _R
    W = _R * C

    tok_flat = tokens.reshape(B, N * C)
    emb_flat = emb.reshape(-1)

    mesh = plsc.VectorSubcoreMesh(
        core_axis_name="c", subcore_axis_name="s",
        num_cores=_NC, num_subcores=_NS)

    @functools.partial(
        pl.kernel,
        out_type=jax.ShapeDtypeStruct((B, N * C), jnp.float32),
        mesh=mesh,
        scratch_types=[
            pltpu.VMEM((W,), jnp.float32),
            pltpu.VMEM((W,), jnp.float32),
            pltpu.VMEM((W,), jnp.float32),
            pltpu.VMEM((W,), jnp.float32),
            pltpu.SemaphoreType.DMA,
            pltpu.SemaphoreType.DMA,
            pltpu.SemaphoreType.DMA,
            pltpu.SemaphoreType.DMA,
            pltpu.SemaphoreType.DMA,
            pltpu.SemaphoreType.DMA,
        ],
    )
    def sc_add(tok_hbm, emb_hbm, out_hbm,
               tv0, tv1, ev0, ev1, sti0, sti1, sto0, sto1, se0, se1):
        tv, ev = [tv0, tv1], [ev0, ev1]
        sti, sto, se = [sti0, sti1], [sto0, sto1], [se0, se1]
        wid = lax.axis_index("s") * _NC + lax.axis_index("c")
        base = wid * (rows_w * C)

        T = n_chunks * B

        def tok_in(t, p):
            chunk, b = divmod(t, B)
            return pltpu.async_copy(
                tok_hbm.at[b, pl.ds(base + chunk * W, W)], tv[p], sti[p])

        emb_dma = [None, None]
        out_dma = [None, None]
        in_dma = [None, None]
        emb_dma[0] = pltpu.async_copy(emb_hbm.at[pl.ds(base, W)], ev[0], se[0])
        in_dma[0] = tok_in(0, 0)

        for t in range(T):
            p = t & 1
            chunk, b = divmod(t, B)
            q = chunk & 1
            if t + 1 < T:
                p1 = (t + 1) & 1
                if out_dma[p1] is not None:
                    out_dma[p1].wait()
                in_dma[p1] = tok_in(t + 1, p1)
            if b == 0 and chunk + 1 < n_chunks:
                q1 = (chunk + 1) & 1
                emb_dma[q1] = pltpu.async_copy(
                    emb_hbm.at[pl.ds(base + (chunk + 1) * W, W)], ev[q1], se[q1])
            in_dma[p].wait()
            if b == 0:
                emb_dma[q].wait()

            tvp, evq = tv[p], ev[q]

            @plsc.parallel_loop(0, W // _L, unroll=8)
            def _(i):
                s = pl.ds(i * _L, _L)
                plsc.addupdate(tvp.at[s], evq[s])

            out_dma[p] = pltpu.async_copy(
                tvp, out_hbm.at[b, pl.ds(base + chunk * W, W)], sto[p])

        out_dma[0].wait()
        out_dma[1].wait()

    out = sc_add(tok_flat, emb_flat)
    return out.reshape(B, N, C)


# speedup vs baseline: 4.3078x; 2.5024x over previous
import functools

import jax
import jax.numpy as jnp
from jax import lax
from jax.experimental import pallas as pl
from jax.experimental.pallas import tpu as pltpu
from jax.experimental.pallas import tpu_sc as plsc

_NC, _NS, _L = 2, 16, 16
_NW = _NC * _NS
_R = 8


def kernel(tokens, emb):
    B, N, C = tokens.shape
    rows_w = N // _NW
    n_chunks = rows_w // _R

    mesh = plsc.VectorSubcoreMesh(
        core_axis_name="c", subcore_axis_name="s",
        num_cores=_NC, num_subcores=_NS)

    @functools.partial(
        pl.kernel,
        out_type=jax.ShapeDtypeStruct((B, N, C), jnp.float32),
        mesh=mesh,
        scratch_types=[
            pltpu.VMEM((_R, C), jnp.float32),
            pltpu.VMEM((_R, C), jnp.float32),
            pltpu.VMEM((_R, C), jnp.float32),
            pltpu.VMEM((_R, C), jnp.float32),
            pltpu.SemaphoreType.DMA,
            pltpu.SemaphoreType.DMA,
            pltpu.SemaphoreType.DMA,
            pltpu.SemaphoreType.DMA,
            pltpu.SemaphoreType.DMA,
            pltpu.SemaphoreType.DMA,
        ],
    )
    def sc_add(tok_hbm, emb_hbm, out_hbm,
               tv0, tv1, ev0, ev1, sti0, sti1, sto0, sto1, se0, se1):
        tv, ev = [tv0, tv1], [ev0, ev1]
        sti, sto, se = [sti0, sti1], [sto0, sto1], [se0, se1]
        wid = lax.axis_index("s") * _NC + lax.axis_index("c")
        base = wid * rows_w

        T = n_chunks * B

        def tok_in(t, p):
            chunk, b = divmod(t, B)
            return pltpu.async_copy(
                tok_hbm.at[b, pl.ds(base + chunk * _R, _R), :], tv[p], sti[p])

        emb_dma = [None, None]
        out_dma = [None, None]
        in_dma = [None, None]
        emb_dma[0] = pltpu.async_copy(
            emb_hbm.at[pl.ds(base, _R), :], ev[0], se[0])
        in_dma[0] = tok_in(0, 0)

        for t in range(T):
            p = t & 1
            chunk, b = divmod(t, B)
            q = chunk & 1
            if t + 1 < T:
                p1 = (t + 1) & 1
                if out_dma[p1] is not None:
                    out_dma[p1].wait()
                in_dma[p1] = tok_in(t + 1, p1)
            if b == 0 and chunk + 1 < n_chunks:
                q1 = (chunk + 1) & 1
                emb_dma[q1] = pltpu.async_copy(
                    emb_hbm.at[pl.ds(base + (chunk + 1) * _R, _R), :],
                    ev[q1], se[q1])
            in_dma[p].wait()
            if b == 0:
                emb_dma[q].wait()

            tvp, evq = tv[p], ev[q]

            @plsc.parallel_loop(0, C // _L, unroll=2)
            def _(j):
                s = pl.ds(j * _L, _L)
                for r in range(_R):
                    plsc.addupdate(tvp.at[r, s], evq[r, s])

            out_dma[p] = pltpu.async_copy(
                tvp, out_hbm.at[b, pl.ds(base + chunk * _R, _R), :], sto[p])

        out_dma[0].wait()
        out_dma[1].wait()

    return sc_add(tokens, emb)


# speedup vs baseline: 4.9690x; 1.1535x over previous
import functools

import jax
import jax.numpy as jnp
from jax import lax
from jax.experimental import pallas as pl
from jax.experimental.pallas import tpu as pltpu
from jax.experimental.pallas import tpu_sc as plsc

_NC, _NS, _L = 2, 16, 16
_NW = _NC * _NS
_R = 8
_NB = 4


def kernel(tokens, emb):
    B, N, C = tokens.shape
    rows_w = N // _NW
    n_chunks = rows_w // _R

    mesh = plsc.VectorSubcoreMesh(
        core_axis_name="c", subcore_axis_name="s",
        num_cores=_NC, num_subcores=_NS)

    @functools.partial(
        pl.kernel,
        out_type=jax.ShapeDtypeStruct((B, N, C), jnp.float32),
        mesh=mesh,
        scratch_types=(
            [pltpu.VMEM((_R, C), jnp.float32) for _ in range(_NB)]
            + [pltpu.VMEM((_R, C), jnp.float32) for _ in range(2)]
            + [pltpu.SemaphoreType.DMA for _ in range(2 * _NB + 2)]
        ),
    )
    def sc_add(tok_hbm, emb_hbm, out_hbm, *refs):
        tv = list(refs[:_NB])
        ev = list(refs[_NB:_NB + 2])
        sti = list(refs[_NB + 2:2 * _NB + 2])
        sto = list(refs[2 * _NB + 2:3 * _NB + 2])
        se = list(refs[3 * _NB + 2:3 * _NB + 4])
        wid = lax.axis_index("s") * _NC + lax.axis_index("c")
        base = wid * rows_w

        T = n_chunks * B

        def tok_in(t):
            chunk, b = divmod(t, B)
            p = t % _NB
            return pltpu.async_copy(
                tok_hbm.at[b, pl.ds(base + chunk * _R, _R), :], tv[p], sti[p])

        def emb_in(chunk):
            q = chunk & 1
            return pltpu.async_copy(
                emb_hbm.at[pl.ds(base + chunk * _R, _R), :], ev[q], se[q])

        emb_dma = [None, None]
        out_dma = [None] * _NB
        in_dma = [None] * _NB
        emb_dma[0] = emb_in(0)
        for k in range(min(_NB - 1, T)):
            in_dma[k] = tok_in(k)

        for t in range(T):
            p = t % _NB
            chunk, b = divmod(t, B)
            q = chunk & 1
            in_dma[p].wait()
            if b == 0:
                emb_dma[q].wait()
                if chunk + 1 < n_chunks:
                    emb_dma[(chunk + 1) & 1] = emb_in(chunk + 1)

            tvp, evq = tv[p], ev[q]

            @plsc.parallel_loop(0, C // _L, unroll=2)
            def _(j):
                s = pl.ds(j * _L, _L)
                for r in range(_R):
                    plsc.addupdate(tvp.at[r, s], evq[r, s])

            out_dma[p] = pltpu.async_copy(
                tvp, out_hbm.at[b, pl.ds(base + chunk * _R, _R), :], sto[p])

            nxt = t + _NB - 1
            if nxt < T:
                pn = nxt % _NB
                if out_dma[pn] is not None:
                    out_dma[pn].wait()
                in_dma[pn] = tok_in(nxt)

        for tl in range(max(0, T - _NB), T):
            out_dma[tl % _NB].wait()

    return sc_add(tokens, emb)


# speedup vs baseline: 5.0064x; 1.0075x over previous
import functools

import jax
import jax.numpy as jnp
from jax import lax
from jax.experimental import pallas as pl
from jax.experimental.pallas import tpu as pltpu
from jax.experimental.pallas import tpu_sc as plsc

_NC, _NS, _L = 2, 16, 16
_NW = _NC * _NS
_R = 8
_NB = 5


def kernel(tokens, emb):
    B, N, C = tokens.shape
    rows_w = N // _NW
    n_chunks = rows_w // _R

    mesh = plsc.VectorSubcoreMesh(
        core_axis_name="c", subcore_axis_name="s",
        num_cores=_NC, num_subcores=_NS)

    @functools.partial(
        pl.kernel,
        out_type=jax.ShapeDtypeStruct((B, N, C), jnp.float32),
        mesh=mesh,
        scratch_types=(
            [pltpu.VMEM((_R, C), jnp.float32) for _ in range(_NB)]
            + [pltpu.VMEM((_R, C), jnp.float32) for _ in range(2)]
            + [pltpu.SemaphoreType.DMA for _ in range(2 * _NB + 2)]
        ),
    )
    def sc_add(tok_hbm, emb_hbm, out_hbm, *refs):
        tv = list(refs[:_NB])
        ev = list(refs[_NB:_NB + 2])
        sti = list(refs[_NB + 2:2 * _NB + 2])
        sto = list(refs[2 * _NB + 2:3 * _NB + 2])
        se = list(refs[3 * _NB + 2:3 * _NB + 4])
        wid = lax.axis_index("s") * _NC + lax.axis_index("c")
        base = wid * rows_w

        T = n_chunks * B

        def tok_in(t):
            chunk, b = divmod(t, B)
            p = t % _NB
            return pltpu.async_copy(
                tok_hbm.at[b, pl.ds(base + chunk * _R, _R), :], tv[p], sti[p])

        def emb_in(chunk):
            q = chunk & 1
            return pltpu.async_copy(
                emb_hbm.at[pl.ds(base + chunk * _R, _R), :], ev[q], se[q])

        emb_dma = [None, None]
        out_dma = [None] * _NB
        in_dma = [None] * _NB
        emb_dma[0] = emb_in(0)
        for k in range(min(_NB - 1, T)):
            in_dma[k] = tok_in(k)

        for t in range(T):
            p = t % _NB
            chunk, b = divmod(t, B)
            q = chunk & 1
            in_dma[p].wait()
            if b == 0:
                emb_dma[q].wait()
                if chunk + 1 < n_chunks:
                    emb_dma[(chunk + 1) & 1] = emb_in(chunk + 1)

            tvp, evq = tv[p], ev[q]

            @plsc.parallel_loop(0, C // _L, unroll=2)
            def _(j):
                s = pl.ds(j * _L, _L)
                for r in range(_R):
                    plsc.addupdate(tvp.at[r, s], evq[r, s])

            out_dma[p] = pltpu.async_copy(
                tvp, out_hbm.at[b, pl.ds(base + chunk * _R, _R), :], sto[p])

            nxt = t + _NB - 1
            if nxt < T:
                pn = nxt % _NB
                if out_dma[pn] is not None:
                    out_dma[pn].wait()
                in_dma[pn] = tok_in(nxt)

        for tl in range(max(0, T - _NB), T):
            out_dma[tl % _NB].wait()

    return sc_add(tokens, emb)


# speedup vs baseline: 5.0153x; 1.0018x over previous
import functools

import jax
import jax.numpy as jnp
from jax import lax
from jax.experimental import pallas as pl
from jax.experimental.pallas import tpu as pltpu
from jax.experimental.pallas import tpu_sc as plsc

_NC, _NS, _L = 2, 16, 16
_NW = _NC * _NS
_R = 8
_NB = 5


def kernel(tokens, emb):
    B, N, C = tokens.shape
    rows_w = N // _NW
    n_chunks = rows_w // _R

    mesh = plsc.VectorSubcoreMesh(
        core_axis_name="c", subcore_axis_name="s",
        num_cores=_NC, num_subcores=_NS)

    @functools.partial(
        pl.kernel,
        out_type=jax.ShapeDtypeStruct((B, N, C), jnp.float32),
        mesh=mesh,
        scratch_types=(
            [pltpu.VMEM((_R, C), jnp.float32) for _ in range(_NB)]
            + [pltpu.VMEM((_R, C), jnp.float32) for _ in range(2)]
            + [pltpu.SemaphoreType.DMA for _ in range(2 * _NB + 2)]
        ),
    )
    def sc_add(tok_hbm, emb_hbm, out_hbm, *refs):
        tv = list(refs[:_NB])
        ev = list(refs[_NB:_NB + 2])
        sti = list(refs[_NB + 2:2 * _NB + 2])
        sto = list(refs[2 * _NB + 2:3 * _NB + 2])
        se = list(refs[3 * _NB + 2:3 * _NB + 4])
        wid = lax.axis_index("s") * _NC + lax.axis_index("c")
        base = wid * rows_w

        T = n_chunks * B

        def tok_in(t):
            chunk, b = divmod(t, B)
            p = t % _NB
            return pltpu.async_copy(
                tok_hbm.at[b, pl.ds(base + chunk * _R, _R), :], tv[p], sti[p])

        def emb_in(chunk):
            q = chunk & 1
            return pltpu.async_copy(
                emb_hbm.at[pl.ds(base + chunk * _R, _R), :], ev[q], se[q])

        emb_dma = [None, None]
        out_dma = [None] * _NB
        in_dma = [None] * _NB
        emb_dma[0] = emb_in(0)
        for k in range(min(_NB - 1, T)):
            in_dma[k] = tok_in(k)

        for t in range(T):
            p = t % _NB
            chunk, b = divmod(t, B)
            q = chunk & 1
            in_dma[p].wait()
            if b == 0:
                emb_dma[q].wait()
                if chunk + 1 < n_chunks:
                    emb_dma[(chunk + 1) & 1] = emb_in(chunk + 1)

            tvp, evq = tv[p], ev[q]

            @plsc.parallel_loop(0, C // _L, unroll=1)
            def _(j):
                s = pl.ds(j * _L, _L)
                for r in range(_R):
                    plsc.addupdate(tvp.at[r, s], evq[r, s])

            out_dma[p] = pltpu.async_copy(
                tvp, out_hbm.at[b, pl.ds(base + chunk * _R, _R), :], sto[p])

            nxt = t + _NB - 1
            if nxt < T:
                pn = nxt % _NB
                if out_dma[pn] is not None:
                    out_dma[pn].wait()
                in_dma[pn] = tok_in(nxt)

        for tl in range(max(0, T - _NB), T):
            out_dma[tl % _NB].wait()

    return sc_add(tokens, emb)


# speedup vs baseline: 5.1279x; 1.0225x over previous
import functools

import jax
import jax.numpy as jnp
from jax import lax
from jax.experimental import pallas as pl
from jax.experimental.pallas import tpu as pltpu
from jax.experimental.pallas import tpu_sc as plsc

_NC, _NS, _L = 2, 16, 16
_NW = _NC * _NS
_R = 4
_NG = 3


def kernel(tokens, emb):
    B, N, C = tokens.shape
    rows_w = N // _NW
    n_groups = rows_w // _R

    mesh = plsc.VectorSubcoreMesh(
        core_axis_name="c", subcore_axis_name="s",
        num_cores=_NC, num_subcores=_NS)

    @functools.partial(
        pl.kernel,
        out_type=jax.ShapeDtypeStruct((B, N, C), jnp.float32),
        mesh=mesh,
        scratch_types=(
            [pltpu.VMEM((_R, C), jnp.float32) for _ in range(_NG * B)]
            + [pltpu.VMEM((_R, C), jnp.float32) for _ in range(2)]
            + [pltpu.SemaphoreType.DMA for _ in range(2 * _NG + 2)]
        ),
    )
    def sc_add(tok_hbm, emb_hbm, out_hbm, *refs):
        tg = [list(refs[k * B:(k + 1) * B]) for k in range(_NG)]
        ev = list(refs[_NG * B:_NG * B + 2])
        sti = list(refs[_NG * B + 2:_NG * B + 2 + _NG])
        sto = list(refs[_NG * B + 2 + _NG:_NG * B + 2 + 2 * _NG])
        se = list(refs[_NG * B + 2 + 2 * _NG:])
        wid = lax.axis_index("s") * _NC + lax.axis_index("c")
        base = wid * rows_w

        def rows(g):
            return pl.ds(base + g * _R, _R)

        def ins(g):
            k = g % _NG
            return [pltpu.async_copy(tok_hbm.at[b, rows(g), :], tg[k][b], sti[k])
                    for b in range(B)]

        def outs(g):
            k = g % _NG
            return [pltpu.async_copy(tg[k][b], out_hbm.at[b, rows(g), :], sto[k])
                    for b in range(B)]

        def emb_in(g):
            return pltpu.async_copy(emb_hbm.at[rows(g), :], ev[g & 1], se[g & 1])

        in_dma = [None] * _NG
        out_dma = [None] * _NG
        emb_dma = [None, None]
        emb_dma[0] = emb_in(0)
        emb_dma[1] = emb_in(1)
        in_dma[0] = ins(0)
        in_dma[1] = ins(1)

        for g in range(n_groups):
            k = g % _NG
            q = g & 1
            if g >= 1 and g + 1 < n_groups:
                emb_dma[(g + 1) & 1] = emb_in(g + 1)
            for d in in_dma[k]:
                d.wait()
            emb_dma[q].wait()

            tgb, evq = tg[k], ev[q]

            @plsc.parallel_loop(0, C // _L, unroll=1)
            def _(j):
                s = pl.ds(j * _L, _L)
                for r in range(_R):
                    e = evq[r, s]
                    for b in range(B):
                        plsc.addupdate(tgb[b].at[r, s], e)

            out_dma[k] = outs(g)

            if g + 2 < n_groups:
                kp = (g + 2) % _NG
                if out_dma[kp] is not None:
                    for d in out_dma[kp]:
                        d.wait()
                in_dma[kp] = ins(g + 2)

        for gl in range(max(0, n_groups - _NG), n_groups):
            for d in out_dma[gl % _NG]:
                d.wait()

    return sc_add(tokens, emb)
